# baseline (speedup 1.0000x reference)
import functools

import jax
import jax.numpy as jnp
from jax import lax
from jax.experimental import pallas as pl
from jax.experimental.pallas import tpu as pltpu
from jax.experimental.pallas import tpu_sc as plsc

D = 128
NUM_CORES = 2
NUM_SUBCORES = 16
NW = NUM_CORES * NUM_SUBCORES
BATCH = 4096
BPW = BATCH // NW
L_T = (20, 50)
L_F = 200
TOT_T = sum(L_T)


def _run_pipeline(ct, k, fire_gather, wait_gather, fire_write, wait_write):
    for b in range(k - 1):
        fire_gather(b, b)
    wait_gather(0)
    fire_write(0, 0)
    fire_gather(k - 1, k - 1)

    main_len = ct - k
    blocks = main_len // k

    def block(gi, _):
        for u in range(k):
            j = 1 + gi * k + u
            b = (1 + u) % k
            bp = u % k
            wait_gather(b)
            fire_write(j, b)
            wait_write(bp)
            fire_gather(j + k - 1, bp)
        return 0

    lax.fori_loop(0, blocks, block, 0)
    for r in range(blocks * k, main_len):
        j = 1 + r
        wait_gather(j % k)
        fire_write(j, j % k)
        wait_write((j - 1) % k)
        fire_gather(j + k - 1, (j - 1) % k)
    for j in range(ct - k + 1, ct):
        wait_gather(j % k)
        fire_write(j, j % k)
    for j in range(ct - k, ct):
        wait_write(j % k)


def _gather_kernel(idx20, idx50, idx200, t20, t50, t200):
    mesh = plsc.VectorSubcoreMesh(core_axis_name="c", subcore_axis_name="s")
    out_type = (
        jax.ShapeDtypeStruct((L_T[0], BATCH, D), jnp.float32),
        jax.ShapeDtypeStruct((L_T[1], BATCH, D), jnp.float32),
        jax.ShapeDtypeStruct((BATCH, L_F, D), jnp.float32),
    )

    @functools.partial(
        pl.kernel,
        out_type=out_type,
        mesh=mesh,
        scratch_types=[
            pltpu.VMEM((TOT_T, BPW), jnp.int32),
            pltpu.VMEM((BPW * L_F,), jnp.int32),
            pltpu.VMEM((L_F, D), jnp.float32),
            pltpu.VMEM((L_F, D), jnp.float32),
            pltpu.VMEM((L_F, D), jnp.float32),
            [pltpu.SemaphoreType.DMA] * 4,
            [pltpu.SemaphoreType.DMA] * 4,
        ],
    )
    def body(i20, i50, i200, tb20, tb50, tb200, o20, o50, o200,
             idx_t, idx_f, rb0, rb1, rb2, gsems, wsems):
        wid = lax.axis_index("s") * NUM_CORES + lax.axis_index("c")
        gsem = tuple(gsems)
        wsem = tuple(wsems)
        base_b = wid * BPW

        tcb = 0
        for idx_hbm, l in ((i20, L_T[0]), (i50, L_T[1])):
            pltpu.sync_copy(idx_hbm.at[:, pl.ds(base_b, BPW)],
                            idx_t.at[pl.ds(tcb, l)])
            tcb += l

        pltpu.sync_copy(i200.at[pl.ds(base_b * L_F, BPW * L_F)], idx_f)

        rows2 = (rb0.at[pl.ds(0, BPW)], rb1.at[pl.ds(0, BPW)],
                 rb2.at[pl.ds(0, BPW)])
        tcb = 0
        for t_hbm, out_hbm, ct in ((tb20, o20, L_T[0]),
                                   (tb50, o50, L_T[1])):

            def fire_gather(j, b, t_hbm=t_hbm, tcb=tcb):
                pltpu.async_copy(t_hbm.at[idx_t.at[tcb + j]],
                                 rows2[b], gsem[b])

            def wait_gather(b, t_hbm=t_hbm, tcb=tcb):
                pltpu.make_async_copy(t_hbm.at[idx_t.at[tcb]],
                                      rows2[b], gsem[b]).wait()

            def fire_write(j, b, out_hbm=out_hbm):
                pltpu.async_copy(rows2[b],
                                 out_hbm.at[j, pl.ds(base_b, BPW)],
                                 wsem[b])

            def wait_write(b, out_hbm=out_hbm):
                pltpu.make_async_copy(rows2[b],
                                      out_hbm.at[0, pl.ds(base_b, BPW)],
                                      wsem[b]).wait()

            _run_pipeline(ct, 3, fire_gather, wait_gather, fire_write,
                          wait_write)
            tcb += ct

        rows3 = (rb0, rb1, rb2)

        h = 96

        def fire_gather_f(j, b):
            pltpu.async_copy(tb200.at[idx_f.at[pl.ds(j * L_F, h)]],
                             rows3[b].at[pl.ds(0, h)], gsem[b])
            pltpu.async_copy(
                tb200.at[idx_f.at[pl.ds(j * L_F + h, L_F - h)]],
                rows3[b].at[pl.ds(h, L_F - h)], gsem[b])

        def wait_gather_f(b):
            pltpu.make_async_copy(tb200.at[idx_f.at[pl.ds(0, h)]],
                                  rows3[b].at[pl.ds(0, h)],
                                  gsem[b]).wait()
            pltpu.make_async_copy(
                tb200.at[idx_f.at[pl.ds(0, L_F - h)]],
                rows3[b].at[pl.ds(h, L_F - h)], gsem[b]).wait()

        def fire_write_f(j, b):
            pltpu.async_copy(rows3[b], o200.at[base_b + j], wsem[b])

        def wait_write_f(b):
            pltpu.make_async_copy(rows3[b], o200.at[base_b], wsem[b]).wait()

        _run_pipeline(BPW, 3, fire_gather_f, wait_gather_f,
                      fire_write_f, wait_write_f)

    return body(idx20, idx50, idx200, t20, t50, t200)


def kernel(idx_20, idx_50, idx_200, table_20, table_50, table_200):
    o20, o50, o200 = _gather_kernel(
        idx_20.T.astype(jnp.int32),
        idx_50.T.astype(jnp.int32),
        idx_200.reshape(-1).astype(jnp.int32),
        table_20, table_50, table_200,
    )
    return (
        jnp.transpose(o20, (1, 0, 2)),
        jnp.transpose(o50, (1, 0, 2)),
        o200,
    )


# speedup vs baseline: 1.0030x; 1.0030x over previous
import functools

import jax
import jax.numpy as jnp
from jax import lax
from jax.experimental import pallas as pl
from jax.experimental.pallas import tpu as pltpu
from jax.experimental.pallas import tpu_sc as plsc

D = 128
NUM_CORES = 2
NUM_SUBCORES = 16
NW = NUM_CORES * NUM_SUBCORES
BATCH = 4096
BPW = BATCH // NW
L_T = (20, 50)
L_F = 200
TOT_T = sum(L_T)


def _run_pipeline(ct, k, fire_gather, wait_gather, fire_write, wait_write):
    for b in range(k - 1):
        fire_gather(b, b)
    wait_gather(0)
    fire_write(0, 0)
    fire_gather(k - 1, k - 1)

    main_len = ct - k
    blocks = main_len // k

    def block(gi, _):
        for u in range(k):
            j = 1 + gi * k + u
            b = (1 + u) % k
            bp = u % k
            wait_gather(b)
            fire_write(j, b)
            wait_write(bp)
            fire_gather(j + k - 1, bp)
        return 0

    lax.fori_loop(0, blocks, block, 0)
    for r in range(blocks * k, main_len):
        j = 1 + r
        wait_gather(j % k)
        fire_write(j, j % k)
        wait_write((j - 1) % k)
        fire_gather(j + k - 1, (j - 1) % k)
    for j in range(ct - k + 1, ct):
        wait_gather(j % k)
        fire_write(j, j % k)
    for j in range(ct - k, ct):
        wait_write(j % k)


def _gather_kernel(idx20, idx50, idx200, t20, t50, t200):
    mesh = plsc.VectorSubcoreMesh(core_axis_name="c", subcore_axis_name="s")
    out_type = (
        jax.ShapeDtypeStruct((L_T[0], BATCH, D), jnp.float32),
        jax.ShapeDtypeStruct((L_T[1], BATCH, D), jnp.float32),
        jax.ShapeDtypeStruct((BATCH, L_F, D), jnp.float32),
    )

    @functools.partial(
        pl.kernel,
        out_type=out_type,
        mesh=mesh,
        scratch_types=[
            pltpu.VMEM((TOT_T, BPW), jnp.int32),
            pltpu.VMEM((BPW * L_F // 2,), jnp.int32),
            pltpu.VMEM((L_F, D), jnp.float32),
            pltpu.VMEM((L_F, D), jnp.float32),
            pltpu.VMEM((L_F, D), jnp.float32),
            pltpu.VMEM((L_F, D), jnp.float32),
            [pltpu.SemaphoreType.DMA] * 4,
            [pltpu.SemaphoreType.DMA] * 4,
        ],
    )
    def body(i20, i50, i200, tb20, tb50, tb200, o20, o50, o200,
             idx_t, idx_f, rb0, rb1, rb2, rb3, gsems, wsems):
        wid = lax.axis_index("s") * NUM_CORES + lax.axis_index("c")
        gsem = tuple(gsems)
        wsem = tuple(wsems)
        base_b = wid * BPW

        tcb = 0
        for idx_hbm, l in ((i20, L_T[0]), (i50, L_T[1])):
            pltpu.sync_copy(idx_hbm.at[:, pl.ds(base_b, BPW)],
                            idx_t.at[pl.ds(tcb, l)])
            tcb += l

        rows2 = (rb0.at[pl.ds(0, BPW)], rb1.at[pl.ds(0, BPW)],
                 rb2.at[pl.ds(0, BPW)], rb3.at[pl.ds(0, BPW)])
        tcb = 0
        for t_hbm, out_hbm, ct in ((tb20, o20, L_T[0]),
                                   (tb50, o50, L_T[1])):

            def fire_gather(j, b, t_hbm=t_hbm, tcb=tcb):
                pltpu.async_copy(t_hbm.at[idx_t.at[tcb + j]],
                                 rows2[b], gsem[b])

            def wait_gather(b, t_hbm=t_hbm, tcb=tcb):
                pltpu.make_async_copy(t_hbm.at[idx_t.at[tcb]],
                                      rows2[b], gsem[b]).wait()

            def fire_write(j, b, out_hbm=out_hbm):
                pltpu.async_copy(rows2[b],
                                 out_hbm.at[j, pl.ds(base_b, BPW)],
                                 wsem[b])

            def wait_write(b, out_hbm=out_hbm):
                pltpu.make_async_copy(rows2[b],
                                      out_hbm.at[0, pl.ds(base_b, BPW)],
                                      wsem[b]).wait()

            _run_pipeline(ct, 4, fire_gather, wait_gather, fire_write,
                          wait_write)
            tcb += ct

        rows4 = (rb0, rb1, rb2, rb3)
        half_b = BPW // 2

        for half in range(2):
            pltpu.sync_copy(
                i200.at[pl.ds((base_b + half * half_b) * L_F,
                              half_b * L_F)], idx_f)

            def fire_gather_f(j, b, half=half):
                pltpu.async_copy(
                    tb200.at[idx_f.at[pl.ds(j * L_F, L_F)]],
                    rows4[b], gsem[b])

            def wait_gather_f(b, half=half):
                pltpu.make_async_copy(
                    tb200.at[idx_f.at[pl.ds(0, L_F)]],
                    rows4[b], gsem[b]).wait()

            def fire_write_f(j, b, half=half):
                pltpu.async_copy(
                    rows4[b],
                    o200.at[base_b + half * half_b + j], wsem[b])

            def wait_write_f(b, half=half):
                pltpu.make_async_copy(
                    rows4[b], o200.at[base_b], wsem[b]).wait()

            _run_pipeline(half_b, 4, fire_gather_f, wait_gather_f,
                          fire_write_f, wait_write_f)

    return body(idx20, idx50, idx200, t20, t50, t200)


def kernel(idx_20, idx_50, idx_200, table_20, table_50, table_200):
    o20, o50, o200 = _gather_kernel(
        idx_20.T.astype(jnp.int32),
        idx_50.T.astype(jnp.int32),
        idx_200.reshape(-1).astype(jnp.int32),
        table_20, table_50, table_200,
    )
    return (
        jnp.transpose(o20, (1, 0, 2)),
        jnp.transpose(o50, (1, 0, 2)),
        o200,
    )
